# baseline (speedup 1.0000x reference)
import jax
import jax.numpy as jnp
from jax import lax
from jax.experimental import pallas as pl
from jax.experimental.pallas import tpu as pltpu
from jax.experimental.pallas import tpu_sc as plsc

VOCAB = 1000000
DIM = 64
MAX_LEN = 200
BATCH = 4096
SCALE = 8.0

NC = 2
NS = 16
NW = NC * NS
SEQ_W = BATCH // NW
NBUF = 4
NSTEP = SEQ_W // NBUF


def _positional_encoding():
    depth = DIM // 2
    positions = jnp.arange(MAX_LEN)[:, None]
    depths = jnp.arange(depth)[None, :] / depth
    angle_rates = 1.0 / (10000.0 ** depths)
    angle_rads = positions * angle_rates
    pos = jnp.concatenate([jnp.sin(angle_rads), jnp.cos(angle_rads)], axis=-1)
    return pos.astype(jnp.float32)


def _tpe_body(x_hbm, pos_hbm, table_hbm, out_hbm,
              idx_all, pos_v, bufs, gsems, wsems):
    wid = lax.axis_index("s") * NC + lax.axis_index("c")
    seq_base = wid * SEQ_W
    pltpu.sync_copy(x_hbm.at[pl.ds(seq_base, SEQ_W)], idx_all)
    pltpu.sync_copy(pos_hbm, pos_v)

    def gather_descs(seq, k):
        descs = []
        for lo, n in ((0, 128), (128, MAX_LEN - 128)):
            descs.append(pltpu.make_async_copy(
                table_hbm.at[idx_all.at[seq, pl.ds(lo, n)]],
                bufs[k].at[pl.ds(lo, n)], gsems[k]))
        return descs

    def wb_desc(seq, k):
        return pltpu.make_async_copy(bufs[k], out_hbm.at[seq_base + seq], wsems[k])

    def compute(k):
        buf = bufs[k]

        def row_body(r, carry):
            for j in range(DIM // 16):
                sl = pl.ds(j * 16, 16)
                buf[r, sl] = buf[r, sl] * SCALE + pos_v[r, sl]
            return carry
        lax.fori_loop(0, MAX_LEN, row_body, 0, unroll=8)

    for k in range(NBUF - 1):
        for d in gather_descs(k, k):
            d.start()

    def step_body(i, carry):
        for k in range(NBUF):
            s = i * NBUF + k
            kn = (k + NBUF - 1) % NBUF

            @pl.when(s + NBUF - 1 < SEQ_W)
            def _():
                @pl.when(s > 0)
                def _():
                    wb_desc(s - 1, kn).wait()
                for d in gather_descs(s + NBUF - 1, kn):
                    d.start()

            for d in gather_descs(s, k):
                d.wait()
            compute(k)
            wb_desc(s, k).start()
        return carry

    lax.fori_loop(0, NSTEP, step_body, 0)

    for k in range(NBUF):
        s = SEQ_W - NBUF + k
        wb_desc(s, (s % NBUF)).wait()


@jax.jit
def kernel(x, token_table):
    pos = _positional_encoding()
    mesh = plsc.VectorSubcoreMesh(core_axis_name="c", subcore_axis_name="s")

    def body(x_hbm, pos_hbm, table_hbm, out_hbm,
             idx_all, pos_v, b0, b1, b2, b3,
             g0, g1, g2, g3, w0, w1, w2, w3):
        _tpe_body(x_hbm, pos_hbm, table_hbm, out_hbm,
                  idx_all, pos_v, (b0, b1, b2, b3),
                  (g0, g1, g2, g3), (w0, w1, w2, w3))

    run = pl.kernel(
        body,
        out_type=jax.ShapeDtypeStruct((BATCH, MAX_LEN, DIM), jnp.float32),
        mesh=mesh,
        scratch_types=[
            pltpu.VMEM((SEQ_W, MAX_LEN), jnp.int32),
            pltpu.VMEM((MAX_LEN, DIM), jnp.float32),
            pltpu.VMEM((MAX_LEN, DIM), jnp.float32),
            pltpu.VMEM((MAX_LEN, DIM), jnp.float32),
            pltpu.VMEM((MAX_LEN, DIM), jnp.float32),
            pltpu.VMEM((MAX_LEN, DIM), jnp.float32),
        ] + [pltpu.SemaphoreType.DMA] * 8,
        compiler_params=pltpu.CompilerParams(use_tc_tiling_on_sc=False),
    )
    return run(x, pos, token_table)


# speedup vs baseline: 1.1469x; 1.1469x over previous
import jax
import jax.numpy as jnp
from jax import lax
from jax.experimental import pallas as pl
from jax.experimental.pallas import tpu as pltpu
from jax.experimental.pallas import tpu_sc as plsc

VOCAB = 1000000
DIM = 64
MAX_LEN = 200
BATCH = 4096
SCALE = 8.0

NC = 2
NS = 16
NW = NC * NS
SEQ_W = BATCH // NW
NCHUNK = SEQ_W // 2
NPAIR = NCHUNK // 2


def _positional_encoding():
    depth = DIM // 2
    positions = jnp.arange(MAX_LEN)[:, None]
    depths = jnp.arange(depth)[None, :] / depth
    angle_rates = 1.0 / (10000.0 ** depths)
    angle_rads = positions * angle_rates
    pos = jnp.concatenate([jnp.sin(angle_rads), jnp.cos(angle_rads)], axis=-1)
    return pos.astype(jnp.float32)


def _gather_descs(table_hbm, idx_all, chunk, buf, sem):
    descs = []
    for si in range(2):
        for lo, n in ((0, 128), (128, MAX_LEN - 128)):
            descs.append(pltpu.make_async_copy(
                table_hbm.at[idx_all.at[2 * chunk + si, pl.ds(lo, n)]],
                buf.at[si, pl.ds(lo, n)], sem))
    return descs


def _fire_gathers(table_hbm, idx_all, chunk, buf, sem):
    for d in _gather_descs(table_hbm, idx_all, chunk, buf, sem):
        d.start()


def _wait_gathers(table_hbm, idx_all, chunk, buf, sem):
    for d in _gather_descs(table_hbm, idx_all, chunk, buf, sem):
        d.wait()


def _tpe_body(x_hbm, pos_hbm, table_hbm, out_hbm,
              idx_all, pos_v, buf_a, buf_b,
              sem_ga, sem_gb, sem_wa, sem_wb):
    wid = lax.axis_index("s") * NC + lax.axis_index("c")
    seq_base = wid * SEQ_W
    pltpu.sync_copy(x_hbm.at[pl.ds(seq_base, SEQ_W)], idx_all)
    pltpu.sync_copy(pos_hbm, pos_v)

    def compute(buf):
        def row_body(r, carry):
            for j in range(DIM // 16):
                sl = pl.ds(j * 16, 16)
                pv = pos_v[r, sl]
                buf[0, r, sl] = buf[0, r, sl] * SCALE + pv
                buf[1, r, sl] = buf[1, r, sl] * SCALE + pv
            return carry
        lax.fori_loop(0, MAX_LEN, row_body, 0, unroll=8)

    def out_slice(chunk):
        return out_hbm.at[pl.ds(seq_base + 2 * chunk, 2)]

    _fire_gathers(table_hbm, idx_all, 0, buf_a, sem_ga)

    def pair_body(c2, carry):
        ca = 2 * c2
        cb = 2 * c2 + 1

        @pl.when(c2 > 0)
        def _():
            pltpu.make_async_copy(buf_b, out_slice(cb - 2), sem_wb).wait()
        _fire_gathers(table_hbm, idx_all, cb, buf_b, sem_gb)

        _wait_gathers(table_hbm, idx_all, ca, buf_a, sem_ga)
        compute(buf_a)
        pltpu.make_async_copy(buf_a, out_slice(ca), sem_wa).start()

        _wait_gathers(table_hbm, idx_all, cb, buf_b, sem_gb)
        compute(buf_b)
        pltpu.make_async_copy(buf_b, out_slice(cb), sem_wb).start()

        pltpu.make_async_copy(buf_a, out_slice(ca), sem_wa).wait()

        @pl.when(c2 + 1 < NPAIR)
        def _():
            _fire_gathers(table_hbm, idx_all, ca + 2, buf_a, sem_ga)
        return carry

    lax.fori_loop(0, NPAIR, pair_body, 0)

    pltpu.make_async_copy(buf_b, out_slice(NCHUNK - 1), sem_wb).wait()


@jax.jit
def kernel(x, token_table):
    pos = _positional_encoding()
    mesh = plsc.VectorSubcoreMesh(core_axis_name="c", subcore_axis_name="s")
    run = pl.kernel(
        _tpe_body,
        out_type=jax.ShapeDtypeStruct((BATCH, MAX_LEN, DIM), jnp.float32),
        mesh=mesh,
        scratch_types=[
            pltpu.VMEM((SEQ_W, MAX_LEN), jnp.int32),
            pltpu.VMEM((MAX_LEN, DIM), jnp.float32),
            pltpu.VMEM((2, MAX_LEN, DIM), jnp.float32),
            pltpu.VMEM((2, MAX_LEN, DIM), jnp.float32),
            pltpu.SemaphoreType.DMA,
            pltpu.SemaphoreType.DMA,
            pltpu.SemaphoreType.DMA,
            pltpu.SemaphoreType.DMA,
        ],
        compiler_params=pltpu.CompilerParams(use_tc_tiling_on_sc=False),
    )
    return run(x, pos, token_table)
